# Initial kernel scaffold; baseline (speedup 1.0000x reference)
import jax
import jax.numpy as jnp
from jax.experimental import pallas as pl


def kernel(nfeats, efeats, edge_index, W_ne1, Wa1, ba1, Wo1, W_ne2, Wa2, ba2, Wo2, Wp, bp):
    raise NotImplementedError("write your pallas kernel here")


# speedup vs baseline: 3.8795x; 3.8795x over previous
import jax
import jax.numpy as jnp
from jax import lax
from jax.experimental import pallas as pl
from jax.experimental.pallas import tpu as pltpu
from jax.experimental.pallas import tpu_sc as plsc

N = 10000
E = 160000
DIN = 256
DE = 16
DOUT = 256

NCORES = 2
NSUB = 16
NW = NCORES * NSUB
EPW = 5120
E_PAD = NW * EPW
CH = 128
NCHUNK = EPW // CH
N_PAD = 10240
RPS = N_PAD // NSUB
NEG = -1e30
F32 = jnp.float32
I32 = jnp.int32

BN = 1000
BE = 8192


def _ee_body(ef_ref, ae_ref, bab_ref, padb_ref, ee_ref):
    ef = ef_ref[...]
    for l in range(2):
        v = jnp.sum(ef * ae_ref[l, :][None, :], axis=1)
        ee_ref[l, :] = v + bab_ref[0, l] + padb_ref[:, 0]


def _ee_call(ef_p, ae, bab, padb):
    return pl.pallas_call(
        _ee_body,
        grid=(E_PAD // BE,),
        in_specs=[
            pl.BlockSpec((BE, DE), lambda i: (i, 0)),
            pl.BlockSpec((2, DE), lambda i: (0, 0)),
            pl.BlockSpec((1, 2), lambda i: (0, 0)),
            pl.BlockSpec((BE, 1), lambda i: (i, 0)),
        ],
        out_specs=pl.BlockSpec((2, BE), lambda i: (0, i)),
        out_shape=jax.ShapeDtypeStruct((2, E_PAD), F32),
    )(ef_p, ae, bab, padb)


def _hm_outs():
    return ([pl.BlockSpec((BN, 128), lambda i: (i, 0)) for _ in range(2)],
            [jax.ShapeDtypeStruct((N, 128), F32) for _ in range(2)])


def _pre_body(h_ref, wneT_ref, a_ref, s12_ref, hm0_ref, hm1_ref):
    h = h_ref[...]
    hm = jnp.dot(h, wneT_ref[...], preferred_element_type=F32)
    hm0_ref[...] = hm[:, :128]
    hm1_ref[...] = hm[:, 128:]
    s12_ref[...] = jnp.dot(h, a_ref[...], preferred_element_type=F32)


def _pre_call(h, wneT, a):
    hm_specs, hm_shapes = _hm_outs()
    return pl.pallas_call(
        _pre_body,
        grid=(N // BN,),
        in_specs=[
            pl.BlockSpec((BN, 256), lambda i: (i, 0)),
            pl.BlockSpec((256, 256), lambda i: (0, 0)),
            pl.BlockSpec((256, 2), lambda i: (0, 0)),
        ],
        out_specs=[pl.BlockSpec((BN, 2), lambda i: (i, 0))] + hm_specs,
        out_shape=[jax.ShapeDtypeStruct((N, 2), F32)] + hm_shapes,
    )(h, wneT, a)


def _zagg_h(hp, g0, g1, us, wneeT, wohT, wozT):
    f = g0[:, :DE] + g1[:, :DE]
    s = g0[:, DE] + g1[:, DE]
    u = jnp.concatenate([us[0] + us[1], us[2] + us[3]], axis=1)
    zagg = (u + jnp.dot(f, wneeT, preferred_element_type=F32)) \
        * (1.0 / (s + 1e-16))[:, None]
    return jnp.maximum(
        jnp.dot(hp, wohT, preferred_element_type=F32)
        + jnp.dot(zagg, wozT, preferred_element_type=F32), 0.0)


def _mid_body(hp_ref, g0_ref, g1_ref, u00_ref, u01_ref, u10_ref, u11_ref,
              wneeT_ref, wohT_ref, wozT_ref, a2_ref, wne2T_ref,
              h_ref, s12_ref, hm0_ref, hm1_ref):
    h = _zagg_h(hp_ref[...], g0_ref[...], g1_ref[...],
                [u00_ref[...], u01_ref[...], u10_ref[...], u11_ref[...]],
                wneeT_ref[...], wohT_ref[...], wozT_ref[...])
    h_ref[...] = h
    s12_ref[...] = jnp.dot(h, a2_ref[...], preferred_element_type=F32)
    hm = jnp.dot(h, wne2T_ref[...], preferred_element_type=F32)
    hm0_ref[...] = hm[:, :128]
    hm1_ref[...] = hm[:, 128:]


def _mid_call(hp, g, u, wneeT, wohT, wozT, a2, wne2T):
    nspec = lambda w: pl.BlockSpec((BN, w), lambda i: (i, 0))
    wspec = lambda r, c: pl.BlockSpec((r, c), lambda i: (0, 0))
    hm_specs, hm_shapes = _hm_outs()
    return pl.pallas_call(
        _mid_body,
        grid=(N // BN,),
        in_specs=[nspec(256), nspec(32), nspec(32),
                  nspec(128), nspec(128), nspec(128), nspec(128),
                  wspec(16, 256), wspec(256, 256), wspec(256, 256),
                  wspec(256, 2), wspec(256, 256)],
        out_specs=[nspec(256), nspec(2)] + hm_specs,
        out_shape=[jax.ShapeDtypeStruct((N, 256), F32),
                   jax.ShapeDtypeStruct((N, 2), F32)] + hm_shapes,
    )(hp, g[0], g[1], u[0, 0], u[0, 1], u[1, 0], u[1, 1],
      wneeT, wohT, wozT, a2, wne2T)


def _fin_body(hp_ref, g0_ref, g1_ref, u00_ref, u01_ref, u10_ref, u11_ref,
              wneeT_ref, wohT_ref, wozT_ref, wp4_ref, bp4_ref, psd_ref):
    h = _zagg_h(hp_ref[...], g0_ref[...], g1_ref[...],
                [u00_ref[...], u01_ref[...], u10_ref[...], u11_ref[...]],
                wneeT_ref[...], wohT_ref[...], wozT_ref[...])
    psd_ref[...] = jnp.dot(h, wp4_ref[...], preferred_element_type=F32) \
        + bp4_ref[...]


def _fin_call(hp, g, u, wneeT, wohT, wozT, wp4, bp4):
    nspec = lambda w: pl.BlockSpec((BN, w), lambda i: (i, 0))
    wspec = lambda r, c: pl.BlockSpec((r, c), lambda i: (0, 0))
    return pl.pallas_call(
        _fin_body,
        grid=(N // BN,),
        in_specs=[nspec(256), nspec(32), nspec(32),
                  nspec(128), nspec(128), nspec(128), nspec(128),
                  wspec(16, 256), wspec(256, 256), wspec(256, 256),
                  wspec(256, 4), wspec(1, 4)],
        out_specs=nspec(4),
        out_shape=jax.ShapeDtypeStruct((N, 4), F32),
    )(hp, g[0], g[1], u[0, 0], u[0, 1], u[1, 0], u[1, 1],
      wneeT, wohT, wozT, wp4, bp4)


_MESH = plsc.VectorSubcoreMesh(core_axis_name="c", subcore_axis_name="s",
                               num_cores=NCORES, num_subcores=NSUB)
_SC_PARAMS = pltpu.CompilerParams(needs_layout_passes=False)


def _sc_ex_body(s12_hbm, src_hbm, dst_hbm, ee_hbm,
                ex_out,
                s12_l, src_l, dst_l, ee_l, exc):
    cid = lax.axis_index("c")
    sid = lax.axis_index("s")
    wid = cid * NSUB + sid
    ebase = wid * EPW

    pltpu.sync_copy(s12_hbm, s12_l)
    pltpu.sync_copy(src_hbm.at[wid], src_l)
    pltpu.sync_copy(dst_hbm.at[wid], dst_l)
    pltpu.sync_copy(ee_hbm.at[pl.ds(ebase, EPW)], ee_l)

    @pl.loop(0, NCHUNK)
    def _chunk(j):
        for g in range(CH // 16):
            off = j * CH + g * 16
            sidx = src_l[j, pl.ds(g * 16, 16)]
            didx = dst_l[j, pl.ds(g * 16, 16)]
            s1v = plsc.load_gather(s12_l, [sidx * 2])
            s2v = plsc.load_gather(s12_l, [didx * 2 + 1])
            ev = s1v + s2v + ee_l[pl.ds(off, 16)]
            ev = jnp.where(ev > 0.0, ev, ev * 0.01)
            exc[pl.ds(g * 16, 16)] = jnp.exp(ev)
        pltpu.sync_copy(exc, ex_out.at[pl.ds(ebase + j * CH, CH)])


def _sc_ex_call(s12, src3, dst3, ee):
    fn = pl.kernel(
        _sc_ex_body,
        out_type=jax.ShapeDtypeStruct((E_PAD,), F32),
        mesh=_MESH,
        compiler_params=_SC_PARAMS,
        scratch_types=[
            pltpu.VMEM((2 * N,), F32),
            pltpu.VMEM((NCHUNK, CH), I32),
            pltpu.VMEM((NCHUNK, CH), I32),
            pltpu.VMEM((EPW,), F32),
            pltpu.VMEM((CH,), F32),
        ],
    )
    return fn(s12, src3, dst3, ee)


def _sc_g_body(dst_hbm, ex_hbm, ef8_hbm, zg_hbm,
               g_out,
               dst_l, ex_l, efc, gbuf, g_sh, sem):
    cid = lax.axis_index("c")
    sid = lax.axis_index("s")
    wid = cid * NSUB + sid
    ebase = wid * EPW
    rbase = sid * RPS

    iota16 = lax.iota(I32, 16)
    sel0 = jnp.where(iota16 == 0, 1.0, 0.0).astype(F32)

    pltpu.sync_copy(dst_hbm.at[wid], dst_l)
    pltpu.sync_copy(ex_hbm.at[pl.ds(ebase, EPW)], ex_l)
    pltpu.sync_copy(zg_hbm.at[pl.ds(rbase, RPS)], g_sh.at[pl.ds(rbase, RPS)])
    plsc.subcore_barrier()

    @pl.loop(0, NCHUNK)
    def _chunk(j):
        idxv = jnp.full((16,), (ebase + j * CH) // 8, I32) + iota16
        pltpu.async_copy(ef8_hbm.at[idxv], efc, sem).wait()
        for r in range(CH):
            exs = plsc.load_gather(ex_l, [jnp.full((16,), j * CH + r, I32)])
            efr = efc[r // 8, pl.ds((r % 8) * 16, 16)]
            gbuf[r, pl.ds(0, 16)] = exs * efr
            gbuf[r, pl.ds(16, 16)] = exs * sel0
        pltpu.sync_copy(gbuf, g_sh.at[dst_l.at[j]], add=True)

    plsc.subcore_barrier()
    pltpu.sync_copy(g_sh.at[pl.ds(rbase, RPS)],
                    g_out.at[cid, pl.ds(rbase, RPS)])


def _sc_g_call(dst3, ex, ef8, zg):
    fn = pl.kernel(
        _sc_g_body,
        out_type=jax.ShapeDtypeStruct((NCORES, N_PAD, 32), F32),
        mesh=_MESH,
        compiler_params=_SC_PARAMS,
        scratch_types=[
            pltpu.VMEM((NCHUNK, CH), I32),
            pltpu.VMEM((EPW,), F32),
            pltpu.VMEM((CH // 8, 128), F32),
            pltpu.VMEM((CH, 32), F32),
            pltpu.VMEM_SHARED((N_PAD, 32), F32),
            pltpu.SemaphoreType.DMA,
        ],
    )
    return fn(dst3, ex, ef8, zg)


def _sc_u_body(src_hbm, dst_hbm, ex_hbm, hm0_hbm, hm1_hbm, zu_hbm,
               u_out,
               src_l, dst_l, ex_l, rows, u_sh, sem):
    cid = lax.axis_index("c")
    sid = lax.axis_index("s")
    wid = cid * NSUB + sid
    ebase = wid * EPW
    rbase = sid * RPS

    pltpu.sync_copy(src_hbm.at[wid], src_l)
    pltpu.sync_copy(dst_hbm.at[wid], dst_l)
    pltpu.sync_copy(ex_hbm.at[pl.ds(ebase, EPW)], ex_l)

    for c, hm_hbm in enumerate((hm0_hbm, hm1_hbm)):
        pltpu.sync_copy(zu_hbm.at[pl.ds(rbase, RPS)],
                        u_sh.at[pl.ds(rbase, RPS)])
        plsc.subcore_barrier()

        @pl.loop(0, NCHUNK)
        def _chunk(j):
            pltpu.async_copy(hm_hbm.at[src_l.at[j]], rows, sem).wait()
            for r in range(CH):
                exs = plsc.load_gather(ex_l, [jnp.full((16,), j * CH + r, I32)])
                for k in range(8):
                    rows[r, pl.ds(k * 16, 16)] = rows[r, pl.ds(k * 16, 16)] * exs
            pltpu.sync_copy(rows, u_sh.at[dst_l.at[j]], add=True)

        plsc.subcore_barrier()
        pltpu.sync_copy(u_sh.at[pl.ds(rbase, RPS)],
                        u_out.at[c, cid, pl.ds(rbase, RPS)])


def _sc_u_call(src3, dst3, ex, hms, zu):
    fn = pl.kernel(
        _sc_u_body,
        out_type=jax.ShapeDtypeStruct((2, NCORES, N_PAD, 128), F32),
        mesh=_MESH,
        compiler_params=_SC_PARAMS,
        scratch_types=[
            pltpu.VMEM((NCHUNK, CH), I32),
            pltpu.VMEM((NCHUNK, CH), I32),
            pltpu.VMEM((EPW,), F32),
            pltpu.VMEM((CH, 128), F32),
            pltpu.VMEM_SHARED((N_PAD, 128), F32),
            pltpu.SemaphoreType.DMA,
        ],
    )
    return fn(src3, dst3, ex, *hms, zu)


def _sc_score_body(psd_hbm, src_hbm, dst_hbm,
                   out0, out1,
                   psd_l, src_l, dst_l, ob0, ob1):
    cid = lax.axis_index("c")
    sid = lax.axis_index("s")
    wid = cid * NSUB + sid
    ebase = wid * EPW

    pltpu.sync_copy(psd_hbm, psd_l)
    pltpu.sync_copy(src_hbm.at[wid], src_l)
    pltpu.sync_copy(dst_hbm.at[wid], dst_l)

    @pl.loop(0, NCHUNK)
    def _chunk(j):
        for g in range(CH // 16):
            sidx = src_l[j, pl.ds(g * 16, 16)] * 4
            didx = dst_l[j, pl.ds(g * 16, 16)] * 4
            v0 = plsc.load_gather(psd_l, [sidx])
            v1 = plsc.load_gather(psd_l, [sidx + 1])
            v2 = plsc.load_gather(psd_l, [didx + 2])
            v3 = plsc.load_gather(psd_l, [didx + 3])
            ob0[pl.ds(g * 16, 16)] = v0 + v2
            ob1[pl.ds(g * 16, 16)] = v1 + v3
        pltpu.sync_copy(ob0, out0.at[pl.ds(ebase + j * CH, CH)])
        pltpu.sync_copy(ob1, out1.at[pl.ds(ebase + j * CH, CH)])


def _sc_score_call(psd, src3, dst3):
    fn = pl.kernel(
        _sc_score_body,
        out_type=[jax.ShapeDtypeStruct((E_PAD,), F32),
                  jax.ShapeDtypeStruct((E_PAD,), F32)],
        mesh=_MESH,
        compiler_params=_SC_PARAMS,
        scratch_types=[
            pltpu.VMEM((4 * N,), F32),
            pltpu.VMEM((NCHUNK, CH), I32),
            pltpu.VMEM((NCHUNK, CH), I32),
            pltpu.VMEM((CH,), F32),
            pltpu.VMEM((CH,), F32),
        ],
    )
    return fn(psd, src3, dst3)


def kernel(nfeats, efeats, edge_index, W_ne1, Wa1, ba1, Wo1,
           W_ne2, Wa2, ba2, Wo2, Wp, bp):
    src = edge_index[0]
    dst = edge_index[1]
    npad = E_PAD - E
    src3 = jnp.concatenate([src, jnp.zeros((npad,), I32)]).reshape(NW, NCHUNK, CH)
    dst3 = jnp.concatenate([dst, jnp.zeros((npad,), I32)]).reshape(NW, NCHUNK, CH)
    ef_p = jnp.concatenate([efeats, jnp.zeros((npad, DE), F32)], axis=0)
    ef8 = ef_p.reshape(E_PAD // 8, 128)
    padb = jnp.where(jnp.arange(E_PAD) < E, 0.0, NEG).astype(F32)[:, None]
    zu = jnp.zeros((N_PAD, 128), F32)
    zg = jnp.zeros((N_PAD, 32), F32)

    ae = jnp.stack([Wa1[0, 2 * DIN:], Wa2[0, 2 * DOUT:]])
    bab = jnp.stack([ba1[0], ba2[0]])[None, :]
    a1 = jnp.stack([Wa1[0, :DIN], Wa1[0, DIN:2 * DIN]], axis=1)
    a2 = jnp.stack([Wa2[0, :DOUT], Wa2[0, DOUT:2 * DOUT]], axis=1)
    wne1T = W_ne1[:, :DIN].T
    wne1eT = W_ne1[:, DIN:].T
    wne2T = W_ne2[:, :DOUT].T
    wne2eT = W_ne2[:, DOUT:].T
    wo1hT = Wo1[:, :DIN].T
    wo1zT = Wo1[:, DIN:].T
    wo2hT = Wo2[:, :DOUT].T
    wo2zT = Wo2[:, DOUT:].T
    wp4 = jnp.concatenate([Wp[:, :DOUT].T, Wp[:, DOUT:].T], axis=1)
    bp4 = jnp.concatenate([bp, jnp.zeros((2,), F32)])[None, :]

    ee = _ee_call(ef_p, ae, bab, padb)

    s12_1, *hms1 = _pre_call(nfeats, wne1T, a1)
    ex1 = _sc_ex_call(s12_1.reshape(2 * N), src3, dst3, ee[0])
    g1 = _sc_g_call(dst3, ex1, ef8, zg)
    u1 = _sc_u_call(src3, dst3, ex1, hms1, zu)
    h1, s12_2, *hms2 = _mid_call(nfeats, g1, u1, wne1eT, wo1hT, wo1zT,
                                 a2, wne2T)
    ex2 = _sc_ex_call(s12_2.reshape(2 * N), src3, dst3, ee[1])
    g2 = _sc_g_call(dst3, ex2, ef8, zg)
    u2 = _sc_u_call(src3, dst3, ex2, hms2, zu)
    psd = _fin_call(h1, g2, u2, wne2eT, wo2hT, wo2zT, wp4, bp4)

    sc0, sc1 = _sc_score_call(psd.reshape(4 * N), src3, dst3)
    return jnp.stack([sc0[:E], sc1[:E]], axis=1)


# speedup vs baseline: 3.8987x; 1.0049x over previous
import jax
import jax.numpy as jnp
from jax import lax
from jax.experimental import pallas as pl
from jax.experimental.pallas import tpu as pltpu
from jax.experimental.pallas import tpu_sc as plsc

N = 10000
E = 160000
DIN = 256
DE = 16
DOUT = 256

NCORES = 2
NSUB = 16
NW = NCORES * NSUB
EPW = 5120
E_PAD = NW * EPW
CH = 128
NCHUNK = EPW // CH
N_PAD = 10240
RPS = N_PAD // NSUB
NEG = -1e30
F32 = jnp.float32
I32 = jnp.int32

BN = 1000
BE = 8192


def _ee_body(ef_ref, ae_ref, bab_ref, padb_ref, ee_ref):
    ef = ef_ref[...]
    for l in range(2):
        v = jnp.sum(ef * ae_ref[l, :][None, :], axis=1)
        ee_ref[l, :] = v + bab_ref[0, l] + padb_ref[:, 0]


def _ee_call(ef_p, ae, bab, padb):
    return pl.pallas_call(
        _ee_body,
        grid=(E_PAD // BE,),
        in_specs=[
            pl.BlockSpec((BE, DE), lambda i: (i, 0)),
            pl.BlockSpec((2, DE), lambda i: (0, 0)),
            pl.BlockSpec((1, 2), lambda i: (0, 0)),
            pl.BlockSpec((BE, 1), lambda i: (i, 0)),
        ],
        out_specs=pl.BlockSpec((2, BE), lambda i: (0, i)),
        out_shape=jax.ShapeDtypeStruct((2, E_PAD), F32),
    )(ef_p, ae, bab, padb)


def _hm_outs():
    return ([pl.BlockSpec((BN, 128), lambda i: (i, 0)) for _ in range(2)],
            [jax.ShapeDtypeStruct((N, 128), F32) for _ in range(2)])


def _pre_body(h_ref, wneT_ref, a_ref, s12_ref, hm0_ref, hm1_ref):
    h = h_ref[...]
    hm = jnp.dot(h, wneT_ref[...], preferred_element_type=F32)
    hm0_ref[...] = hm[:, :128]
    hm1_ref[...] = hm[:, 128:]
    s12_ref[...] = jnp.dot(h, a_ref[...], preferred_element_type=F32)


def _pre_call(h, wneT, a):
    hm_specs, hm_shapes = _hm_outs()
    return pl.pallas_call(
        _pre_body,
        grid=(N // BN,),
        in_specs=[
            pl.BlockSpec((BN, 256), lambda i: (i, 0)),
            pl.BlockSpec((256, 256), lambda i: (0, 0)),
            pl.BlockSpec((256, 2), lambda i: (0, 0)),
        ],
        out_specs=[pl.BlockSpec((BN, 2), lambda i: (i, 0))] + hm_specs,
        out_shape=[jax.ShapeDtypeStruct((N, 2), F32)] + hm_shapes,
    )(h, wneT, a)


def _zagg_h(hp, g0, g1, us, wneeT, wohT, wozT):
    f = g0[:, :DE] + g1[:, :DE]
    s = g0[:, DE] + g1[:, DE]
    u = jnp.concatenate([us[0] + us[1], us[2] + us[3]], axis=1)
    zagg = (u + jnp.dot(f, wneeT, preferred_element_type=F32)) \
        * (1.0 / (s + 1e-16))[:, None]
    return jnp.maximum(
        jnp.dot(hp, wohT, preferred_element_type=F32)
        + jnp.dot(zagg, wozT, preferred_element_type=F32), 0.0)


def _mid_body(hp_ref, g0_ref, g1_ref, u00_ref, u01_ref, u10_ref, u11_ref,
              wneeT_ref, wohT_ref, wozT_ref, a2_ref, wne2T_ref,
              h_ref, s12_ref, hm0_ref, hm1_ref):
    h = _zagg_h(hp_ref[...], g0_ref[...], g1_ref[...],
                [u00_ref[...], u01_ref[...], u10_ref[...], u11_ref[...]],
                wneeT_ref[...], wohT_ref[...], wozT_ref[...])
    h_ref[...] = h
    s12_ref[...] = jnp.dot(h, a2_ref[...], preferred_element_type=F32)
    hm = jnp.dot(h, wne2T_ref[...], preferred_element_type=F32)
    hm0_ref[...] = hm[:, :128]
    hm1_ref[...] = hm[:, 128:]


def _mid_call(hp, g, u, wneeT, wohT, wozT, a2, wne2T):
    nspec = lambda w: pl.BlockSpec((BN, w), lambda i: (i, 0))
    wspec = lambda r, c: pl.BlockSpec((r, c), lambda i: (0, 0))
    hm_specs, hm_shapes = _hm_outs()
    return pl.pallas_call(
        _mid_body,
        grid=(N // BN,),
        in_specs=[nspec(256), nspec(32), nspec(32),
                  nspec(128), nspec(128), nspec(128), nspec(128),
                  wspec(16, 256), wspec(256, 256), wspec(256, 256),
                  wspec(256, 2), wspec(256, 256)],
        out_specs=[nspec(256), nspec(2)] + hm_specs,
        out_shape=[jax.ShapeDtypeStruct((N, 256), F32),
                   jax.ShapeDtypeStruct((N, 2), F32)] + hm_shapes,
    )(hp, g[0], g[1], u[0, 0], u[0, 1], u[1, 0], u[1, 1],
      wneeT, wohT, wozT, a2, wne2T)


def _fin_body(hp_ref, g0_ref, g1_ref, u00_ref, u01_ref, u10_ref, u11_ref,
              wneeT_ref, wohT_ref, wozT_ref, wp4_ref, bp4_ref, psd_ref):
    h = _zagg_h(hp_ref[...], g0_ref[...], g1_ref[...],
                [u00_ref[...], u01_ref[...], u10_ref[...], u11_ref[...]],
                wneeT_ref[...], wohT_ref[...], wozT_ref[...])
    psd_ref[...] = jnp.dot(h, wp4_ref[...], preferred_element_type=F32) \
        + bp4_ref[...]


def _fin_call(hp, g, u, wneeT, wohT, wozT, wp4, bp4):
    nspec = lambda w: pl.BlockSpec((BN, w), lambda i: (i, 0))
    wspec = lambda r, c: pl.BlockSpec((r, c), lambda i: (0, 0))
    return pl.pallas_call(
        _fin_body,
        grid=(N // BN,),
        in_specs=[nspec(256), nspec(32), nspec(32),
                  nspec(128), nspec(128), nspec(128), nspec(128),
                  wspec(16, 256), wspec(256, 256), wspec(256, 256),
                  wspec(256, 4), wspec(1, 4)],
        out_specs=nspec(4),
        out_shape=jax.ShapeDtypeStruct((N, 4), F32),
    )(hp, g[0], g[1], u[0, 0], u[0, 1], u[1, 0], u[1, 1],
      wneeT, wohT, wozT, wp4, bp4)


_MESH = plsc.VectorSubcoreMesh(core_axis_name="c", subcore_axis_name="s",
                               num_cores=NCORES, num_subcores=NSUB)
_SC_PARAMS = pltpu.CompilerParams(needs_layout_passes=False)


def _sc_ex_body(s12_hbm, src_hbm, dst_hbm, ee_hbm,
                ex_out,
                s12_l, src_l, dst_l, ee_l, exc):
    cid = lax.axis_index("c")
    sid = lax.axis_index("s")
    wid = cid * NSUB + sid
    ebase = wid * EPW

    pltpu.sync_copy(s12_hbm, s12_l)
    pltpu.sync_copy(src_hbm.at[wid], src_l)
    pltpu.sync_copy(dst_hbm.at[wid], dst_l)
    pltpu.sync_copy(ee_hbm.at[pl.ds(ebase, EPW)], ee_l)

    @pl.loop(0, NCHUNK)
    def _chunk(j):
        for g in range(CH // 16):
            off = j * CH + g * 16
            sidx = src_l[j, pl.ds(g * 16, 16)]
            didx = dst_l[j, pl.ds(g * 16, 16)]
            s1v = plsc.load_gather(s12_l, [sidx * 2])
            s2v = plsc.load_gather(s12_l, [didx * 2 + 1])
            ev = s1v + s2v + ee_l[pl.ds(off, 16)]
            ev = jnp.where(ev > 0.0, ev, ev * 0.01)
            exc[pl.ds(off, 16)] = jnp.exp(ev)

    pltpu.sync_copy(exc, ex_out.at[pl.ds(ebase, EPW)])


def _sc_ex_call(s12, src3, dst3, ee):
    fn = pl.kernel(
        _sc_ex_body,
        out_type=jax.ShapeDtypeStruct((E_PAD,), F32),
        mesh=_MESH,
        compiler_params=_SC_PARAMS,
        scratch_types=[
            pltpu.VMEM((2 * N,), F32),
            pltpu.VMEM((NCHUNK, CH), I32),
            pltpu.VMEM((NCHUNK, CH), I32),
            pltpu.VMEM((EPW,), F32),
            pltpu.VMEM((EPW,), F32),
        ],
    )
    return fn(s12, src3, dst3, ee)


def _sc_g_body(dst_hbm, ex_hbm, ef8_hbm, zg_hbm,
               g_out,
               dst_l, ex_l, efc, gbuf, g_sh, sem):
    cid = lax.axis_index("c")
    sid = lax.axis_index("s")
    wid = cid * NSUB + sid
    ebase = wid * EPW
    rbase = sid * RPS

    iota16 = lax.iota(I32, 16)
    sel0 = jnp.where(iota16 == 0, 1.0, 0.0).astype(F32)

    pltpu.sync_copy(dst_hbm.at[wid], dst_l)
    pltpu.sync_copy(ex_hbm.at[pl.ds(ebase, EPW)], ex_l)
    pltpu.sync_copy(zg_hbm.at[pl.ds(rbase, RPS)], g_sh.at[pl.ds(rbase, RPS)])
    plsc.subcore_barrier()

    @pl.loop(0, NCHUNK)
    def _chunk(j):
        idxv = jnp.full((16,), (ebase + j * CH) // 8, I32) + iota16
        pltpu.async_copy(ef8_hbm.at[idxv], efc, sem).wait()
        for r in range(CH):
            exs = plsc.load_gather(ex_l, [jnp.full((16,), j * CH + r, I32)])
            efr = efc[r // 8, pl.ds((r % 8) * 16, 16)]
            gbuf[r, pl.ds(0, 16)] = exs * efr
            gbuf[r, pl.ds(16, 16)] = exs * sel0
        pltpu.sync_copy(gbuf, g_sh.at[dst_l.at[j]], add=True)

    plsc.subcore_barrier()
    pltpu.sync_copy(g_sh.at[pl.ds(rbase, RPS)],
                    g_out.at[cid, pl.ds(rbase, RPS)])


def _sc_g_call(dst3, ex, ef8, zg):
    fn = pl.kernel(
        _sc_g_body,
        out_type=jax.ShapeDtypeStruct((NCORES, N_PAD, 32), F32),
        mesh=_MESH,
        compiler_params=_SC_PARAMS,
        scratch_types=[
            pltpu.VMEM((NCHUNK, CH), I32),
            pltpu.VMEM((EPW,), F32),
            pltpu.VMEM((CH // 8, 128), F32),
            pltpu.VMEM((CH, 32), F32),
            pltpu.VMEM_SHARED((N_PAD, 32), F32),
            pltpu.SemaphoreType.DMA,
        ],
    )
    return fn(dst3, ex, ef8, zg)


def _sc_u_body(src_hbm, dst_hbm, ex_hbm, hm0_hbm, hm1_hbm, zu_hbm,
               u_out,
               src_l, dst_l, ex_l, rows, u_sh, sem):
    cid = lax.axis_index("c")
    sid = lax.axis_index("s")
    wid = cid * NSUB + sid
    ebase = wid * EPW
    rbase = sid * RPS

    pltpu.sync_copy(src_hbm.at[wid], src_l)
    pltpu.sync_copy(dst_hbm.at[wid], dst_l)
    pltpu.sync_copy(ex_hbm.at[pl.ds(ebase, EPW)], ex_l)

    for c, hm_hbm in enumerate((hm0_hbm, hm1_hbm)):
        pltpu.sync_copy(zu_hbm.at[pl.ds(rbase, RPS)],
                        u_sh.at[pl.ds(rbase, RPS)])
        plsc.subcore_barrier()

        @pl.loop(0, NCHUNK)
        def _chunk(j):
            pltpu.async_copy(hm_hbm.at[src_l.at[j]], rows, sem).wait()
            for r in range(CH):
                exs = plsc.load_gather(ex_l, [jnp.full((16,), j * CH + r, I32)])
                for k in range(8):
                    rows[r, pl.ds(k * 16, 16)] = rows[r, pl.ds(k * 16, 16)] * exs
            pltpu.sync_copy(rows, u_sh.at[dst_l.at[j]], add=True)

        plsc.subcore_barrier()
        pltpu.sync_copy(u_sh.at[pl.ds(rbase, RPS)],
                        u_out.at[c, cid, pl.ds(rbase, RPS)])


def _sc_u_call(src3, dst3, ex, hms, zu):
    fn = pl.kernel(
        _sc_u_body,
        out_type=jax.ShapeDtypeStruct((2, NCORES, N_PAD, 128), F32),
        mesh=_MESH,
        compiler_params=_SC_PARAMS,
        scratch_types=[
            pltpu.VMEM((NCHUNK, CH), I32),
            pltpu.VMEM((NCHUNK, CH), I32),
            pltpu.VMEM((EPW,), F32),
            pltpu.VMEM((CH, 128), F32),
            pltpu.VMEM_SHARED((N_PAD, 128), F32),
            pltpu.SemaphoreType.DMA,
        ],
    )
    return fn(src3, dst3, ex, *hms, zu)


def _sc_score_body(psd_hbm, src_hbm, dst_hbm,
                   out0, out1,
                   psd_l, src_l, dst_l, ob0, ob1):
    cid = lax.axis_index("c")
    sid = lax.axis_index("s")
    wid = cid * NSUB + sid
    ebase = wid * EPW

    pltpu.sync_copy(psd_hbm, psd_l)
    pltpu.sync_copy(src_hbm.at[wid], src_l)
    pltpu.sync_copy(dst_hbm.at[wid], dst_l)

    @pl.loop(0, NCHUNK)
    def _chunk(j):
        for g in range(CH // 16):
            sidx = src_l[j, pl.ds(g * 16, 16)] * 4
            didx = dst_l[j, pl.ds(g * 16, 16)] * 4
            v0 = plsc.load_gather(psd_l, [sidx])
            v1 = plsc.load_gather(psd_l, [sidx + 1])
            v2 = plsc.load_gather(psd_l, [didx + 2])
            v3 = plsc.load_gather(psd_l, [didx + 3])
            ob0[pl.ds(g * 16, 16)] = v0 + v2
            ob1[pl.ds(g * 16, 16)] = v1 + v3
        pltpu.sync_copy(ob0, out0.at[pl.ds(ebase + j * CH, CH)])
        pltpu.sync_copy(ob1, out1.at[pl.ds(ebase + j * CH, CH)])


def _sc_score_call(psd, src3, dst3):
    fn = pl.kernel(
        _sc_score_body,
        out_type=[jax.ShapeDtypeStruct((E_PAD,), F32),
                  jax.ShapeDtypeStruct((E_PAD,), F32)],
        mesh=_MESH,
        compiler_params=_SC_PARAMS,
        scratch_types=[
            pltpu.VMEM((4 * N,), F32),
            pltpu.VMEM((NCHUNK, CH), I32),
            pltpu.VMEM((NCHUNK, CH), I32),
            pltpu.VMEM((CH,), F32),
            pltpu.VMEM((CH,), F32),
        ],
    )
    return fn(psd, src3, dst3)


def kernel(nfeats, efeats, edge_index, W_ne1, Wa1, ba1, Wo1,
           W_ne2, Wa2, ba2, Wo2, Wp, bp):
    src = edge_index[0]
    dst = edge_index[1]
    npad = E_PAD - E
    src3 = jnp.concatenate([src, jnp.zeros((npad,), I32)]).reshape(NW, NCHUNK, CH)
    dst3 = jnp.concatenate([dst, jnp.zeros((npad,), I32)]).reshape(NW, NCHUNK, CH)
    ef_p = jnp.concatenate([efeats, jnp.zeros((npad, DE), F32)], axis=0)
    ef8 = ef_p.reshape(E_PAD // 8, 128)
    padb = jnp.where(jnp.arange(E_PAD) < E, 0.0, NEG).astype(F32)[:, None]
    zu = jnp.zeros((N_PAD, 128), F32)
    zg = jnp.zeros((N_PAD, 32), F32)

    ae = jnp.stack([Wa1[0, 2 * DIN:], Wa2[0, 2 * DOUT:]])
    bab = jnp.stack([ba1[0], ba2[0]])[None, :]
    a1 = jnp.stack([Wa1[0, :DIN], Wa1[0, DIN:2 * DIN]], axis=1)
    a2 = jnp.stack([Wa2[0, :DOUT], Wa2[0, DOUT:2 * DOUT]], axis=1)
    wne1T = W_ne1[:, :DIN].T
    wne1eT = W_ne1[:, DIN:].T
    wne2T = W_ne2[:, :DOUT].T
    wne2eT = W_ne2[:, DOUT:].T
    wo1hT = Wo1[:, :DIN].T
    wo1zT = Wo1[:, DIN:].T
    wo2hT = Wo2[:, :DOUT].T
    wo2zT = Wo2[:, DOUT:].T
    wp4 = jnp.concatenate([Wp[:, :DOUT].T, Wp[:, DOUT:].T], axis=1)
    bp4 = jnp.concatenate([bp, jnp.zeros((2,), F32)])[None, :]

    ee = _ee_call(ef_p, ae, bab, padb)

    s12_1, *hms1 = _pre_call(nfeats, wne1T, a1)
    ex1 = _sc_ex_call(s12_1.reshape(2 * N), src3, dst3, ee[0])
    g1 = _sc_g_call(dst3, ex1, ef8, zg)
    u1 = _sc_u_call(src3, dst3, ex1, hms1, zu)
    h1, s12_2, *hms2 = _mid_call(nfeats, g1, u1, wne1eT, wo1hT, wo1zT,
                                 a2, wne2T)
    ex2 = _sc_ex_call(s12_2.reshape(2 * N), src3, dst3, ee[1])
    g2 = _sc_g_call(dst3, ex2, ef8, zg)
    u2 = _sc_u_call(src3, dst3, ex2, hms2, zu)
    psd = _fin_call(h1, g2, u2, wne2eT, wo2hT, wo2zT, wp4, bp4)

    sc0, sc1 = _sc_score_call(psd.reshape(4 * N), src3, dst3)
    return jnp.stack([sc0[:E], sc1[:E]], axis=1)
